# Initial kernel scaffold; baseline (speedup 1.0000x reference)
import jax
import jax.numpy as jnp
from jax.experimental import pallas as pl


def kernel(X, color_table, pos_table):
    raise NotImplementedError("write your pallas kernel here")


# speedup vs baseline: 2.8968x; 2.8968x over previous
import functools

import jax
import jax.numpy as jnp
from jax import lax
from jax.experimental import pallas as pl
from jax.experimental.pallas import tpu as pltpu
from jax.experimental.pallas import tpu_sc as plsc

_NC = 2
_NS = 16
_NW = _NC * _NS
_K = 128
_D = 64
_V = 7


def _gather_body(nchunks, ids_hbm, table_hbm, out_hbm, idx_v, rows_v, sem):
    wid = lax.axis_index("s") * _NC + lax.axis_index("c")
    pltpu.sync_copy(ids_hbm.at[pl.ds(wid * nchunks, nchunks)], idx_v)
    row_base = wid * nchunks * _K

    def step(c, carry):
        pltpu.async_copy(table_hbm.at[idx_v.at[c]], rows_v, sem).wait()
        pltpu.sync_copy(rows_v, out_hbm.at[pl.ds(row_base + c * _K, _K)])
        return carry

    lax.fori_loop(0, nchunks, step, 0)


def kernel(X, color_table, pos_table):
    del pos_table
    batch, seq = X.shape
    nb = seq // 5
    xr = X.reshape(batch, nb, 5).astype(jnp.int32)
    pids = jnp.stack(
        [xr[:, :, 1] * _V + xr[:, :, 2], xr[:, :, 3] * _V + xr[:, :, 4]],
        axis=-1,
    ).reshape(-1)
    pair_table = jnp.concatenate(
        [
            jnp.broadcast_to(color_table[:, None, :], (_V, _V, _D)),
            jnp.broadcast_to(color_table[None, :, :], (_V, _V, _D)),
        ],
        axis=-1,
    ).reshape(_V * _V, 2 * _D)

    g = pids.shape[0]
    nchunks = g // (_NW * _K)
    ids2d = pids.reshape(_NW * nchunks, _K)

    body = functools.partial(_gather_body, nchunks)
    out2d = pl.kernel(
        body,
        out_type=jax.ShapeDtypeStruct((g, 2 * _D), jnp.float32),
        mesh=plsc.VectorSubcoreMesh(core_axis_name="c", subcore_axis_name="s"),
        scratch_types=[
            pltpu.VMEM((nchunks, _K), jnp.int32),
            pltpu.VMEM((_K, 2 * _D), jnp.float32),
            pltpu.SemaphoreType.DMA,
        ],
    )(ids2d, pair_table)
    return out2d.reshape(batch, nb * 4 * _D)


# speedup vs baseline: 2.9258x; 1.0100x over previous
import functools

import jax
import jax.numpy as jnp
from jax import lax
from jax.experimental import pallas as pl
from jax.experimental.pallas import tpu as pltpu
from jax.experimental.pallas import tpu_sc as plsc

_NC = 2
_NS = 16
_NW = _NC * _NS
_K = 128
_D = 64
_V = 7


_NBUF = 4


def _gather_body(nchunks, ids_hbm, table_hbm, out_hbm, idx_v,
                 *bufs_and_sems):
    rows = bufs_and_sems[:_NBUF]
    gsem = bufs_and_sems[_NBUF:2 * _NBUF]
    wsem = bufs_and_sems[2 * _NBUF:3 * _NBUF]
    wid = lax.axis_index("s") * _NC + lax.axis_index("c")
    pltpu.sync_copy(ids_hbm.at[pl.ds(wid * nchunks, nchunks)], idx_v)
    row_base = wid * nchunks * _K

    def gather_start(c, b):
        pltpu.async_copy(table_hbm.at[idx_v.at[c]], rows[b], gsem[b])

    def gather_wait(b):
        pltpu.make_async_copy(table_hbm.at[idx_v.at[0]], rows[b],
                              gsem[b]).wait()

    def write_start(c, b):
        pltpu.async_copy(rows[b],
                         out_hbm.at[pl.ds(row_base + c * _K, _K)], wsem[b])

    def write_wait(b):
        pltpu.make_async_copy(rows[b], out_hbm.at[pl.ds(row_base, _K)],
                              wsem[b]).wait()

    for c0 in range(_NBUF - 1):
        gather_start(c0, c0)

    def step(cc, carry):
        for b in range(_NBUF):
            c = cc * _NBUF + b
            gather_wait(b)
            nxt = c + _NBUF - 1
            nb = (b + _NBUF - 1) % _NBUF

            @pl.when(nxt < nchunks)
            def _():
                @pl.when(c >= 1)
                def _():
                    write_wait(nb)

                gather_start(nxt, nb)

            write_start(c, b)
        return carry

    lax.fori_loop(0, nchunks // _NBUF, step, 0)
    for b in range(_NBUF):
        write_wait(b)


def kernel(X, color_table, pos_table):
    del pos_table
    batch, seq = X.shape
    nb = seq // 5
    xr = X.reshape(batch, nb, 5).astype(jnp.int32)
    pids = jnp.stack(
        [xr[:, :, 1] * _V + xr[:, :, 2], xr[:, :, 3] * _V + xr[:, :, 4]],
        axis=-1,
    ).reshape(-1)
    pair_table = jnp.concatenate(
        [
            jnp.broadcast_to(color_table[:, None, :], (_V, _V, _D)),
            jnp.broadcast_to(color_table[None, :, :], (_V, _V, _D)),
        ],
        axis=-1,
    ).reshape(_V * _V, 2 * _D)

    g = pids.shape[0]
    nchunks = g // (_NW * _K)
    ids2d = pids.reshape(_NW * nchunks, _K)

    body = functools.partial(_gather_body, nchunks)
    out2d = pl.kernel(
        body,
        out_type=jax.ShapeDtypeStruct((g, 2 * _D), jnp.float32),
        mesh=plsc.VectorSubcoreMesh(core_axis_name="c", subcore_axis_name="s"),
        scratch_types=(
            [pltpu.VMEM((nchunks, _K), jnp.int32)]
            + [pltpu.VMEM((_K, 2 * _D), jnp.float32)] * _NBUF
            + [pltpu.SemaphoreType.DMA] * (2 * _NBUF)
        ),
    )(ids2d, pair_table)
    return out2d.reshape(batch, nb * 4 * _D)


# speedup vs baseline: 7.2957x; 2.4936x over previous
import functools

import jax
import jax.numpy as jnp
from jax import lax
from jax.experimental import pallas as pl
from jax.experimental.pallas import tpu as pltpu
from jax.experimental.pallas import tpu_sc as plsc

_NC = 2
_NS = 16
_NW = _NC * _NS
_K = 128
_D = 64
_V = 7


_NBUF = 4


def _gather_body(nchunks, ids_hbm, table_hbm, out_hbm, idx_v, table_v,
                 *bufs_and_sems):
    rows = bufs_and_sems[:_NBUF]
    gsem = bufs_and_sems[_NBUF:2 * _NBUF]
    wsem = bufs_and_sems[2 * _NBUF:3 * _NBUF]
    wid = lax.axis_index("s") * _NC + lax.axis_index("c")
    pltpu.sync_copy(table_hbm, table_v)
    pltpu.sync_copy(ids_hbm.at[pl.ds(wid * nchunks, nchunks)], idx_v)
    row_base = wid * nchunks * _K

    def gather_start(c, b):
        pltpu.async_copy(table_v.at[idx_v.at[c]], rows[b], gsem[b])

    def gather_wait(b):
        pltpu.make_async_copy(table_v.at[idx_v.at[0]], rows[b],
                              gsem[b]).wait()

    def write_start(c, b):
        pltpu.async_copy(rows[b],
                         out_hbm.at[pl.ds(row_base + c * _K, _K)], wsem[b])

    def write_wait(b):
        pltpu.make_async_copy(rows[b], out_hbm.at[pl.ds(row_base, _K)],
                              wsem[b]).wait()

    for c0 in range(_NBUF - 1):
        gather_start(c0, c0)

    def step(cc, carry):
        for b in range(_NBUF):
            c = cc * _NBUF + b
            gather_wait(b)
            nxt = c + _NBUF - 1
            nb = (b + _NBUF - 1) % _NBUF

            @pl.when(nxt < nchunks)
            def _():
                @pl.when(c >= 1)
                def _():
                    write_wait(nb)

                gather_start(nxt, nb)

            write_start(c, b)
        return carry

    lax.fori_loop(0, nchunks // _NBUF, step, 0)
    for b in range(_NBUF):
        write_wait(b)


def kernel(X, color_table, pos_table):
    del pos_table
    batch, seq = X.shape
    nb = seq // 5
    xr = X.reshape(batch, nb, 5).astype(jnp.int32)
    pids = jnp.stack(
        [xr[:, :, 1] * _V + xr[:, :, 2], xr[:, :, 3] * _V + xr[:, :, 4]],
        axis=-1,
    ).reshape(-1)
    pair_table = jnp.concatenate(
        [
            jnp.broadcast_to(color_table[:, None, :], (_V, _V, _D)),
            jnp.broadcast_to(color_table[None, :, :], (_V, _V, _D)),
        ],
        axis=-1,
    ).reshape(_V * _V, 2 * _D)

    g = pids.shape[0]
    nchunks = g // (_NW * _K)
    ids2d = pids.reshape(_NW * nchunks, _K)

    body = functools.partial(_gather_body, nchunks)
    out2d = pl.kernel(
        body,
        out_type=jax.ShapeDtypeStruct((g, 2 * _D), jnp.float32),
        mesh=plsc.VectorSubcoreMesh(core_axis_name="c", subcore_axis_name="s"),
        scratch_types=(
            [pltpu.VMEM((nchunks, _K), jnp.int32),
             pltpu.VMEM_SHARED((_V * _V, 2 * _D), jnp.float32)]
            + [pltpu.VMEM((_K, 2 * _D), jnp.float32)] * _NBUF
            + [pltpu.SemaphoreType.DMA] * (2 * _NBUF)
        ),
    )(ids2d, pair_table)
    return out2d.reshape(batch, nb * 4 * _D)
